# baseline (speedup 1.0000x reference)
import functools

import jax
import jax.numpy as jnp
from jax import lax
from jax.experimental import pallas as pl
from jax.experimental.pallas import tpu as pltpu
from jax.experimental.pallas import tpu_sc as plsc

B = 16
N = 100000
P = 18
PG = 9
HALF = N // 2
C = 2000
NCH = HALF // C
VPC = C // 16
HPP = 256
HPL = P * HPP
HTOT = 16 * HPL
ACCW = 80
ACCT = 16 * ACCW
PASSES = ((None, 23, 255, 8), (23, 15, 255, 8), (15, 7, 255, 8), (7, 0, 127, 7))


def _zero_ref(ref, nrows):
    z16 = jnp.zeros((16,), jnp.float32)

    @plsc.parallel_loop(0, nrows, 1, unroll=4)
    def _(j):
        base = j * 256
        for t in range(16):
            ref[pl.ds(base + t * 16, 16)] = z16


def _lane_tree_reduce(ref, row_words):
    for step in (8, 4, 2, 1):
        nrows = step * row_words // 16

        @plsc.parallel_loop(0, nrows, 1, unroll=4)
        def _(j, step=step):
            off = j * 16
            a = ref[pl.ds(off, 16)]
            bb = ref[pl.ds(step * row_words + off, 16)]
            ref[pl.ds(off, 16)] = a + bb


def _sc_body(xyz_hbm, seg_hbm, out_hbm, s_hbm,
             xyzbuf0, xyzbuf1, segbuf0, segbuf1, sbuf0, sbuf1,
             hist, acc, cumbuf, mergebuf, mergebuf_i,
             cxt, cyt, czt, cntt, kremt, preft, medt, outrow,
             histshm, accshm, prefshm, medshm,
             sem0, sem1, stsem0, stsem1):
    cidx = lax.axis_index("c")
    sidx = lax.axis_index("s")
    wid = cidx * 16 + sidx
    b = wid // 2
    g = wid % 2
    plo = g * PG
    qlo = (1 - g) * PG
    psidx = sidx ^ 1

    xyzbufs = (xyzbuf0, xyzbuf1)
    segbufs = (segbuf0, segbuf1)
    sbufs = (sbuf0, sbuf1)
    sems = (sem0, sem1)
    stsems = (stsem0, stsem1)

    lane = lax.iota(jnp.int32, 16)
    lane80 = lane * ACCW
    lane_hist = lane * HPL
    lane33 = lane * 33
    zeros_i = jnp.zeros((16,), jnp.int32)
    ones_f = jnp.zeros((16,), jnp.float32) + 1.0
    lane0_mask = lane == 0

    def splat(x):
        return zeros_i + x

    def extract(vec, idx_vec):
        return jnp.sum(jnp.where(lane == idx_vec, vec, jnp.zeros((16,), vec.dtype)))

    base0 = b * N + g * HALF

    def xyzseg_pairs(k, slot):
        base = base0 + k * C
        xb = xyzbufs[slot]
        return [
            (xyz_hbm.at[pl.ds(base, C)], xb.at[pl.ds(0, C)]),
            (xyz_hbm.at[pl.ds(B * N + base, C)], xb.at[pl.ds(C, C)]),
            (xyz_hbm.at[pl.ds(2 * B * N + base, C)], xb.at[pl.ds(2 * C, C)]),
            (seg_hbm.at[pl.ds(base, C)], segbufs[slot]),
        ]

    def segs_pairs(k, slot):
        base = base0 + k * C
        return [
            (seg_hbm.at[pl.ds(base, C)], segbufs[slot]),
            (s_hbm.at[pl.ds(base, C)], sbufs[slot]),
        ]

    def _start(pairs, sem):
        for src, dst in pairs:
            pltpu.async_copy(src, dst, sem)

    def _wait(pairs, sem):
        for src, dst in pairs:
            pltpu.make_async_copy(src, dst, sem).wait()

    def stream_scan(make_pairs, compute):
        _start(make_pairs(0, 0), sems[0])

        def body(j, carry):
            k0 = 2 * j
            k1 = 2 * j + 1
            _start(make_pairs(k1, 1), sems[1])
            _wait(make_pairs(k0, 0), sems[0])
            compute(k0, 0)
            _start(make_pairs(k1 + 1, 0), sems[0])
            _wait(make_pairs(k1, 1), sems[1])
            compute(k1, 1)
            return carry

        lax.fori_loop(0, NCH // 2, body, 0)
        _wait(make_pairs(NCH - 1, 0), sems[0])
        compute(NCH - 1, 0)

    def merge_group_table(tab, shm, mb32):
        pltpu.sync_copy(tab.at[pl.ds(0, 32)], shm.at[pl.ds(sidx * 32, 32)])
        plsc.subcore_barrier()
        pltpu.sync_copy(shm.at[pl.ds(psidx * 32, 32)], mb32)
        m0 = (lane >= qlo) & (lane < qlo + PG)
        m1 = ((lane + 16) >= qlo) & ((lane + 16) < qlo + PG)
        own0 = tab[pl.ds(0, 16)]
        own1 = tab[pl.ds(16, 16)]
        pv0 = mb32[pl.ds(0, 16)]
        pv1 = mb32[pl.ds(16, 16)]
        tab[pl.ds(0, 16)] = jnp.where(m0, pv0, own0)
        tab[pl.ds(16, 16)] = jnp.where(m1, pv1, own1)

    def merge_acc():
        _lane_tree_reduce(acc, ACCW)
        pltpu.sync_copy(acc.at[pl.ds(0, ACCW)], accshm.at[pl.ds(sidx * ACCW, ACCW)])
        plsc.subcore_barrier()
        mb = mergebuf.at[pl.ds(0, ACCW)]
        pltpu.sync_copy(accshm.at[pl.ds(psidx * ACCW, ACCW)], mb)
        for t in range(ACCW // 16):
            a = acc[pl.ds(t * 16, 16)]
            acc[pl.ds(t * 16, 16)] = a + mb[pl.ds(t * 16, 16)]

    def rebroadcast(tab):
        v0 = tab[pl.ds(0, 16)]
        v1 = tab[pl.ds(16, 16)]
        for l in range(1, 16):
            tab[pl.ds(l * 33, 16)] = v0
            tab[pl.ds(l * 33 + 16, 16)] = v1

    neg1_i = splat(-1)
    neg1_f = jnp.zeros((16,), jnp.float32) - 1.0
    for l in range(16):
        preft[pl.ds(l * 33, 16)] = neg1_i
        preft[pl.ds(l * 33 + 16, 16)] = neg1_i
        medt[pl.ds(l * 33, 16)] = neg1_f
        medt[pl.ds(l * 33 + 16, 16)] = neg1_f

    _zero_ref(acc, ACCT // 256)

    def s1_compute(k, slot):
        segb = segbufs[slot]
        xb = xyzbufs[slot]

        @plsc.parallel_loop(0, VPC, 1, unroll=4)
        def _(i):
            sv = segb[pl.ds(i * 16, 16)]
            xv = xb[pl.ds(i * 16, 16)]
            yv = xb[pl.ds(C + i * 16, 16)]
            zv = xb[pl.ds(2 * C + i * 16, 16)]
            ai = lane80 + sv * 4
            plsc.addupdate_scatter(acc, [ai], xv)
            plsc.addupdate_scatter(acc, [ai + 1], yv)
            plsc.addupdate_scatter(acc, [ai + 2], zv)
            plsc.addupdate_scatter(acc, [ai + 3], ones_f)

    stream_scan(xyzseg_pairs, s1_compute)
    merge_acc()

    for base, off in ((0, 0), (16, 64)):
        i4 = lane * 4 + off
        sx = plsc.load_gather(acc, [i4])
        sy = plsc.load_gather(acc, [i4 + 1])
        sz = plsc.load_gather(acc, [i4 + 2])
        cn = plsc.load_gather(acc, [i4 + 3])
        safe = jnp.maximum(cn, 1.0)
        cxt[pl.ds(base, 16)] = sx / safe
        cyt[pl.ds(base, 16)] = sy / safe
        czt[pl.ds(base, 16)] = sz / safe
        cntt[pl.ds(base, 16)] = cn
        cni = cn.astype(jnp.int32)
        kv = jnp.maximum((cni - 1) >> 1, 0).astype(jnp.float32)
        kremt[pl.ds(base, 16)] = kv
    rebroadcast(cxt)
    rebroadcast(cyt)
    rebroadcast(czt)

    def merge_hist_and_select(pass_idx, width):
        _lane_tree_reduce(hist, HPL)
        pltpu.sync_copy(hist.at[pl.ds(0, HPL)],
                        histshm.at[pl.ds(sidx * HPL, HPL)])
        plsc.subcore_barrier()
        mb = mergebuf.at[pl.ds(0, PG * HPP)]
        pltpu.sync_copy(
            histshm.at[pl.ds(psidx * HPL + plo * HPP, PG * HPP)], mb)

        @plsc.parallel_loop(0, PG * HPP // 16, 1, unroll=4)
        def _(t):
            off = plo * HPP + t * 16
            a = hist[pl.ds(off, 16)]
            hist[pl.ds(off, 16)] = a + mb[pl.ds(t * 16, 16)]

        def part_body(p, carry):
            rb = (plo + p) * HPP
            for t in range(16):
                hv = hist[pl.ds(rb + t * 16, 16)]
                cumbuf[pl.ds(t * 16, 16)] = plsc.cumsum(hv)
            gt = plsc.load_gather(cumbuf, [lane * 16 + 15])
            gc = plsc.cumsum(gt)
            ge = gc - gt
            pgidx = splat(plo + p)
            kv = plsc.load_gather(kremt, [pgidx])
            tgt = kv + 1.0
            gsel = jnp.minimum(plsc.all_reduce_ffs(gc >= tgt), 15)
            wg = plsc.load_gather(cumbuf, [gsel * 16 + lane])
            bg = extract(ge, gsel)
            dsel = jnp.minimum(plsc.all_reduce_ffs((bg + wg) >= tgt), 15)
            digit = gsel * 16 + dsel
            below = bg + extract(wg, dsel - 1)
            if pass_idx == 0:
                prefnew = digit
            else:
                pv = plsc.load_gather(preft, [lane33 + pgidx])
                prefnew = (pv << width) | digit
            plsc.store_scatter(kremt, [pgidx], kv - below, mask=lane0_mask)
            plsc.store_scatter(preft, [lane33 + pgidx], prefnew)
            if pass_idx == len(PASSES) - 1:
                plsc.store_scatter(medt, [lane33 + pgidx],
                                   plsc.bitcast(prefnew, jnp.float32))
            return carry

        lax.fori_loop(0, PG, part_body, 0)
        if pass_idx == len(PASSES) - 1:
            merge_group_table(medt, medshm, mergebuf.at[pl.ds(0, 32)])
            rebroadcast(medt)
        else:
            merge_group_table(preft, prefshm, mergebuf_i)
            rebroadcast(preft)

    _zero_ref(hist, HTOT // 256)

    def s2_compute(k, slot):
        segb = segbufs[slot]
        xb = xyzbufs[slot]
        sb = sbufs[slot]

        @pl.when(k >= 2)
        def _():
            pltpu.make_async_copy(
                sb, s_hbm.at[pl.ds(base0 + (k - 2) * C, C)],
                stsems[slot]).wait()

        @plsc.parallel_loop(0, VPC, 1, unroll=4)
        def _(i):
            sv = segb[pl.ds(i * 16, 16)]
            xv = xb[pl.ds(i * 16, 16)]
            yv = xb[pl.ds(C + i * 16, 16)]
            zv = xb[pl.ds(2 * C + i * 16, 16)]
            dx = xv - plsc.load_gather(cxt, [lane33 + sv])
            dy = yv - plsc.load_gather(cyt, [lane33 + sv])
            dz = zv - plsc.load_gather(czt, [lane33 + sv])
            s = dx * dx + dy * dy + dz * dz
            sb[pl.ds(i * 16, 16)] = s
            u = plsc.bitcast(s, jnp.int32)
            bucket = u >> 23
            hidx = lane_hist + sv * HPP + bucket
            plsc.addupdate_scatter(hist, [hidx], ones_f)

        pltpu.async_copy(sb, s_hbm.at[pl.ds(base0 + k * C, C)], stsems[slot])

    stream_scan(xyzseg_pairs, s2_compute)
    for kk in (NCH - 2, NCH - 1):
        pltpu.make_async_copy(
            sbufs[kk % 2], s_hbm.at[pl.ds(base0 + kk * C, C)],
            stsems[kk % 2]).wait()

    merge_hist_and_select(0, PASSES[0][3])

    for pass_idx in range(1, len(PASSES)):
        ps, bs, bm, width = PASSES[pass_idx]
        _zero_ref(hist, HTOT // 256)

        def sp_compute(k, slot, ps=ps, bs=bs, bm=bm):
            segb = segbufs[slot]
            sb = sbufs[slot]

            @plsc.parallel_loop(0, VPC, 1, unroll=4)
            def _(i):
                sv = segb[pl.ds(i * 16, 16)]
                s = sb[pl.ds(i * 16, 16)]
                u = plsc.bitcast(s, jnp.int32)
                pv = plsc.load_gather(preft, [lane33 + sv])
                m = (u >> ps) == pv
                bucket = (u >> bs) & bm
                hidx = lane_hist + sv * HPP + bucket
                plsc.addupdate_scatter(hist, [hidx], ones_f, mask=m)

        stream_scan(segs_pairs, sp_compute)
        merge_hist_and_select(pass_idx, width)

    _zero_ref(acc, ACCT // 256)

    def s6_compute(k, slot):
        segb = segbufs[slot]
        xb = xyzbufs[slot]

        @plsc.parallel_loop(0, VPC, 1, unroll=4)
        def _(i):
            sv = segb[pl.ds(i * 16, 16)]
            xv = xb[pl.ds(i * 16, 16)]
            yv = xb[pl.ds(C + i * 16, 16)]
            zv = xb[pl.ds(2 * C + i * 16, 16)]
            dx = xv - plsc.load_gather(cxt, [lane33 + sv])
            dy = yv - plsc.load_gather(cyt, [lane33 + sv])
            dz = zv - plsc.load_gather(czt, [lane33 + sv])
            s = dx * dx + dy * dy + dz * dz
            medv = plsc.load_gather(medt, [lane33 + sv])
            fm = s <= medv
            ai = lane80 + sv * 4
            plsc.addupdate_scatter(acc, [ai], xv, mask=fm)
            plsc.addupdate_scatter(acc, [ai + 1], yv, mask=fm)
            plsc.addupdate_scatter(acc, [ai + 2], zv, mask=fm)
            plsc.addupdate_scatter(acc, [ai + 3], ones_f, mask=fm)

    stream_scan(xyzseg_pairs, s6_compute)
    merge_acc()

    pg = plo + lane
    lm = lane < PG
    i4 = jnp.where(lm, pg * 4, 0)
    fsx = plsc.load_gather(acc, [i4])
    fsy = plsc.load_gather(acc, [i4 + 1])
    fsz = plsc.load_gather(acc, [i4 + 2])
    fcn = plsc.load_gather(acc, [i4 + 3])
    cxv = plsc.load_gather(cxt, [pg])
    cyv = plsc.load_gather(cyt, [pg])
    czv = plsc.load_gather(czt, [pg])
    cnv = plsc.load_gather(cntt, [pg])
    den = jnp.maximum(fcn, 1.0)
    zf = jnp.zeros((16,), jnp.float32)
    has_f = fcn > 0.0
    has_c = cnv > 0.0
    bx = jnp.where(has_c, jnp.where(has_f, fsx / den, cxv), zf)
    by = jnp.where(has_c, jnp.where(has_f, fsy / den, cyv), zf)
    bz = jnp.where(has_c, jnp.where(has_f, fsz / den, czv), zf)
    for t in range(4):
        outrow[pl.ds(t * 16, 16)] = zf
    oidx = jnp.where(lm, lane * 3, 48)
    plsc.store_scatter(outrow, [oidx], bx, mask=lm)
    plsc.store_scatter(outrow, [oidx + 1], by, mask=lm)
    plsc.store_scatter(outrow, [oidx + 2], bz, mask=lm)
    pltpu.sync_copy(outrow, out_hbm.at[pl.ds(wid * 64, 64)])


@functools.lru_cache(maxsize=1)
def _build():
    mesh = plsc.VectorSubcoreMesh(core_axis_name="c", subcore_axis_name="s",
                                  num_cores=2, num_subcores=16)
    return pl.kernel(
        _sc_body,
        out_type=(
            jax.ShapeDtypeStruct((32 * 64,), jnp.float32),
            jax.ShapeDtypeStruct((B * N,), jnp.float32),
        ),
        mesh=mesh,
        compiler_params=pltpu.CompilerParams(needs_layout_passes=False),
        scratch_types=[
            pltpu.VMEM((C * 3,), jnp.float32),
            pltpu.VMEM((C * 3,), jnp.float32),
            pltpu.VMEM((C,), jnp.int32),
            pltpu.VMEM((C,), jnp.int32),
            pltpu.VMEM((C,), jnp.float32),
            pltpu.VMEM((C,), jnp.float32),
            pltpu.VMEM((HTOT,), jnp.float32),
            pltpu.VMEM((ACCT,), jnp.float32),
            pltpu.VMEM((256,), jnp.float32),
            pltpu.VMEM((PG * HPP,), jnp.float32),
            pltpu.VMEM((32,), jnp.int32),
            pltpu.VMEM((528,), jnp.float32),
            pltpu.VMEM((528,), jnp.float32),
            pltpu.VMEM((528,), jnp.float32),
            pltpu.VMEM((32,), jnp.float32),
            pltpu.VMEM((32,), jnp.float32),
            pltpu.VMEM((528,), jnp.int32),
            pltpu.VMEM((528,), jnp.float32),
            pltpu.VMEM((64,), jnp.float32),
            pltpu.VMEM_SHARED((16 * HPL,), jnp.float32),
            pltpu.VMEM_SHARED((16 * ACCW,), jnp.float32),
            pltpu.VMEM_SHARED((16 * 32,), jnp.int32),
            pltpu.VMEM_SHARED((16 * 32,), jnp.float32),
            pltpu.SemaphoreType.DMA,
            pltpu.SemaphoreType.DMA,
            pltpu.SemaphoreType.DMA,
            pltpu.SemaphoreType.DMA,
        ],
        name="pose_sc",
    )


def kernel(xyz, seg_labels):
    xyz_planar = jnp.transpose(xyz, (2, 0, 1)).reshape(-1)
    rows, _ = _build()(xyz_planar, seg_labels.reshape(-1))
    return rows.reshape(32, 64)[:, :27].reshape(B, 2, PG, 3).reshape(B, P, 3)


# speedup vs baseline: 1.0400x; 1.0400x over previous
import functools

import jax
import jax.numpy as jnp
from jax import lax
from jax.experimental import pallas as pl
from jax.experimental.pallas import tpu as pltpu
from jax.experimental.pallas import tpu_sc as plsc

B = 16
N = 100000
P = 18
PG = 9
HALF = N // 2
C = 2000
NCH = HALF // C
VPC = C // 16
HPP = 256
HPL = P * HPP
HTOT = 16 * HPL
ACCW = 80
ACCT = 16 * ACCW
PASSES = ((None, 23, 255, 8), (23, 15, 255, 8), (15, 7, 255, 8), (7, 0, 127, 7))


def _zero_ref(ref, nrows):
    z16 = jnp.zeros((16,), jnp.float32)

    @plsc.parallel_loop(0, nrows, 1, unroll=4)
    def _(j):
        base = j * 256
        for t in range(16):
            ref[pl.ds(base + t * 16, 16)] = z16


def _lane_tree_reduce(ref, row_words):
    for step in (8, 4, 2, 1):
        nrows = step * row_words // 16

        @plsc.parallel_loop(0, nrows, 1, unroll=4)
        def _(j, step=step):
            off = j * 16
            a = ref[pl.ds(off, 16)]
            bb = ref[pl.ds(step * row_words + off, 16)]
            ref[pl.ds(off, 16)] = a + bb


def _sc_body(xyz_hbm, seg_hbm, out_hbm, s_hbm,
             xyzbuf0, xyzbuf1, segbuf0, segbuf1, sbuf0, sbuf1,
             hist, acc, cumbuf, mergebuf, mergebuf_i,
             cxt, cyt, czt, cntt, kremt, preft, medt, outrow,
             histshm, accshm, prefshm, medshm,
             sem0, sem1, stsem0, stsem1):
    cidx = lax.axis_index("c")
    sidx = lax.axis_index("s")
    wid = cidx * 16 + sidx
    b = wid // 2
    g = wid % 2
    plo = g * PG
    qlo = (1 - g) * PG
    psidx = sidx ^ 1

    xyzbufs = (xyzbuf0, xyzbuf1)
    segbufs = (segbuf0, segbuf1)
    sbufs = (sbuf0, sbuf1)
    sems = (sem0, sem1)
    stsems = (stsem0, stsem1)

    lane = lax.iota(jnp.int32, 16)
    lane80 = lane * ACCW
    lane_hist = lane * HPL
    zeros_i = jnp.zeros((16,), jnp.int32)
    ones_f = jnp.zeros((16,), jnp.float32) + 1.0
    lane0_mask = lane == 0

    def splat(x):
        return zeros_i + x

    def extract(vec, idx_vec):
        return jnp.sum(jnp.where(lane == idx_vec, vec, jnp.zeros((16,), vec.dtype)))

    base0 = b * N + g * HALF

    def xyzseg_pairs(k, slot):
        base = base0 + k * C
        xb = xyzbufs[slot]
        return [
            (xyz_hbm.at[pl.ds(base, C)], xb.at[pl.ds(0, C)]),
            (xyz_hbm.at[pl.ds(B * N + base, C)], xb.at[pl.ds(C, C)]),
            (xyz_hbm.at[pl.ds(2 * B * N + base, C)], xb.at[pl.ds(2 * C, C)]),
            (seg_hbm.at[pl.ds(base, C)], segbufs[slot]),
        ]

    def segs_pairs(k, slot):
        base = base0 + k * C
        return [
            (seg_hbm.at[pl.ds(base, C)], segbufs[slot]),
            (s_hbm.at[pl.ds(base, C)], sbufs[slot]),
        ]

    def _start(pairs, sem):
        for src, dst in pairs:
            pltpu.async_copy(src, dst, sem)

    def _wait(pairs, sem):
        for src, dst in pairs:
            pltpu.make_async_copy(src, dst, sem).wait()

    def stream_scan(make_pairs, compute):
        _start(make_pairs(0, 0), sems[0])

        def body(j, carry):
            k0 = 2 * j
            k1 = 2 * j + 1
            _start(make_pairs(k1, 1), sems[1])
            _wait(make_pairs(k0, 0), sems[0])
            compute(k0, 0)
            _start(make_pairs(k1 + 1, 0), sems[0])
            _wait(make_pairs(k1, 1), sems[1])
            compute(k1, 1)
            return carry

        lax.fori_loop(0, NCH // 2, body, 0)
        _wait(make_pairs(NCH - 1, 0), sems[0])
        compute(NCH - 1, 0)

    def merge_group_table(tab, shm, mb32):
        pltpu.sync_copy(tab.at[pl.ds(0, 32)], shm.at[pl.ds(sidx * 32, 32)])
        plsc.subcore_barrier()
        pltpu.sync_copy(shm.at[pl.ds(psidx * 32, 32)], mb32)
        m0 = (lane >= qlo) & (lane < qlo + PG)
        m1 = ((lane + 16) >= qlo) & ((lane + 16) < qlo + PG)
        own0 = tab[pl.ds(0, 16)]
        own1 = tab[pl.ds(16, 16)]
        pv0 = mb32[pl.ds(0, 16)]
        pv1 = mb32[pl.ds(16, 16)]
        tab[pl.ds(0, 16)] = jnp.where(m0, pv0, own0)
        tab[pl.ds(16, 16)] = jnp.where(m1, pv1, own1)

    def merge_acc():
        _lane_tree_reduce(acc, ACCW)
        pltpu.sync_copy(acc.at[pl.ds(0, ACCW)], accshm.at[pl.ds(sidx * ACCW, ACCW)])
        plsc.subcore_barrier()
        mb = mergebuf.at[pl.ds(0, ACCW)]
        pltpu.sync_copy(accshm.at[pl.ds(psidx * ACCW, ACCW)], mb)
        for t in range(ACCW // 16):
            a = acc[pl.ds(t * 16, 16)]
            acc[pl.ds(t * 16, 16)] = a + mb[pl.ds(t * 16, 16)]

    neg1_i = splat(-1)
    neg1_f = jnp.zeros((16,), jnp.float32) - 1.0
    preft[pl.ds(0, 16)] = neg1_i
    preft[pl.ds(16, 16)] = neg1_i
    medt[pl.ds(0, 16)] = neg1_f
    medt[pl.ds(16, 16)] = neg1_f

    _zero_ref(acc, ACCT // 256)

    def s1_compute(k, slot):
        segb = segbufs[slot]
        xb = xyzbufs[slot]

        @plsc.parallel_loop(0, VPC, 1, unroll=4)
        def _(i):
            sv = segb[pl.ds(i * 16, 16)]
            xv = xb[pl.ds(i * 16, 16)]
            yv = xb[pl.ds(C + i * 16, 16)]
            zv = xb[pl.ds(2 * C + i * 16, 16)]
            ai = lane80 + sv * 4
            plsc.addupdate_scatter(acc, [ai], xv)
            plsc.addupdate_scatter(acc, [ai + 1], yv)
            plsc.addupdate_scatter(acc, [ai + 2], zv)
            plsc.addupdate_scatter(acc, [ai + 3], ones_f)

    stream_scan(xyzseg_pairs, s1_compute)
    merge_acc()

    for base, off in ((0, 0), (16, 64)):
        i4 = lane * 4 + off
        sx = plsc.load_gather(acc, [i4])
        sy = plsc.load_gather(acc, [i4 + 1])
        sz = plsc.load_gather(acc, [i4 + 2])
        cn = plsc.load_gather(acc, [i4 + 3])
        safe = jnp.maximum(cn, 1.0)
        cxt[pl.ds(base, 16)] = sx / safe
        cyt[pl.ds(base, 16)] = sy / safe
        czt[pl.ds(base, 16)] = sz / safe
        cntt[pl.ds(base, 16)] = cn
        cni = cn.astype(jnp.int32)
        kv = jnp.maximum((cni - 1) >> 1, 0).astype(jnp.float32)
        kremt[pl.ds(base, 16)] = kv

    def merge_hist_and_select(pass_idx, width):
        _lane_tree_reduce(hist, HPL)
        pltpu.sync_copy(hist.at[pl.ds(0, HPL)],
                        histshm.at[pl.ds(sidx * HPL, HPL)])
        plsc.subcore_barrier()
        mb = mergebuf.at[pl.ds(0, PG * HPP)]
        pltpu.sync_copy(
            histshm.at[pl.ds(psidx * HPL + plo * HPP, PG * HPP)], mb)

        @plsc.parallel_loop(0, PG * HPP // 16, 1, unroll=4)
        def _(t):
            off = plo * HPP + t * 16
            a = hist[pl.ds(off, 16)]
            hist[pl.ds(off, 16)] = a + mb[pl.ds(t * 16, 16)]

        def part_body(p, carry):
            rb = (plo + p) * HPP
            for t in range(16):
                hv = hist[pl.ds(rb + t * 16, 16)]
                cumbuf[pl.ds(t * 16, 16)] = plsc.cumsum(hv)
            gt = plsc.load_gather(cumbuf, [lane * 16 + 15])
            gc = plsc.cumsum(gt)
            ge = gc - gt
            pgidx = splat(plo + p)
            kv = plsc.load_gather(kremt, [pgidx])
            tgt = kv + 1.0
            gsel = jnp.minimum(plsc.all_reduce_ffs(gc >= tgt), 15)
            wg = plsc.load_gather(cumbuf, [gsel * 16 + lane])
            bg = extract(ge, gsel)
            dsel = jnp.minimum(plsc.all_reduce_ffs((bg + wg) >= tgt), 15)
            digit = gsel * 16 + dsel
            below = bg + extract(wg, dsel - 1)
            if pass_idx == 0:
                prefnew = digit
            else:
                pv = plsc.load_gather(preft, [pgidx])
                prefnew = (pv << width) | digit
            plsc.store_scatter(kremt, [pgidx], kv - below, mask=lane0_mask)
            plsc.store_scatter(preft, [pgidx], prefnew, mask=lane0_mask)
            if pass_idx == len(PASSES) - 1:
                plsc.store_scatter(medt, [pgidx],
                                   plsc.bitcast(prefnew, jnp.float32),
                                   mask=lane0_mask)
            return carry

        lax.fori_loop(0, PG, part_body, 0)
        if pass_idx == len(PASSES) - 1:
            merge_group_table(medt, medshm, mergebuf.at[pl.ds(0, 32)])
        else:
            merge_group_table(preft, prefshm, mergebuf_i)

    _zero_ref(hist, HTOT // 256)

    def s2_compute(k, slot):
        segb = segbufs[slot]
        xb = xyzbufs[slot]
        sb = sbufs[slot]

        @pl.when(k >= 2)
        def _():
            pltpu.make_async_copy(
                sb, s_hbm.at[pl.ds(base0 + (k - 2) * C, C)],
                stsems[slot]).wait()

        @plsc.parallel_loop(0, VPC, 1, unroll=4)
        def _(i):
            sv = segb[pl.ds(i * 16, 16)]
            xv = xb[pl.ds(i * 16, 16)]
            yv = xb[pl.ds(C + i * 16, 16)]
            zv = xb[pl.ds(2 * C + i * 16, 16)]
            dx = xv - plsc.load_gather(cxt, [sv])
            dy = yv - plsc.load_gather(cyt, [sv])
            dz = zv - plsc.load_gather(czt, [sv])
            s = dx * dx + dy * dy + dz * dz
            sb[pl.ds(i * 16, 16)] = s
            u = plsc.bitcast(s, jnp.int32)
            bucket = u >> 23
            hidx = lane_hist + sv * HPP + bucket
            plsc.addupdate_scatter(hist, [hidx], ones_f)

        pltpu.async_copy(sb, s_hbm.at[pl.ds(base0 + k * C, C)], stsems[slot])

    stream_scan(xyzseg_pairs, s2_compute)
    for kk in (NCH - 2, NCH - 1):
        pltpu.make_async_copy(
            sbufs[kk % 2], s_hbm.at[pl.ds(base0 + kk * C, C)],
            stsems[kk % 2]).wait()

    merge_hist_and_select(0, PASSES[0][3])

    for pass_idx in range(1, len(PASSES)):
        ps, bs, bm, width = PASSES[pass_idx]
        _zero_ref(hist, HTOT // 256)

        def sp_compute(k, slot, ps=ps, bs=bs, bm=bm):
            segb = segbufs[slot]
            sb = sbufs[slot]

            @plsc.parallel_loop(0, VPC, 1, unroll=4)
            def _(i):
                sv = segb[pl.ds(i * 16, 16)]
                s = sb[pl.ds(i * 16, 16)]
                u = plsc.bitcast(s, jnp.int32)
                pv = plsc.load_gather(preft, [sv])
                m = (u >> ps) == pv
                bucket = (u >> bs) & bm
                hidx = lane_hist + sv * HPP + bucket
                plsc.addupdate_scatter(hist, [hidx], ones_f, mask=m)

        stream_scan(segs_pairs, sp_compute)
        merge_hist_and_select(pass_idx, width)

    _zero_ref(acc, ACCT // 256)

    def s6_compute(k, slot):
        segb = segbufs[slot]
        xb = xyzbufs[slot]

        @plsc.parallel_loop(0, VPC, 1, unroll=4)
        def _(i):
            sv = segb[pl.ds(i * 16, 16)]
            xv = xb[pl.ds(i * 16, 16)]
            yv = xb[pl.ds(C + i * 16, 16)]
            zv = xb[pl.ds(2 * C + i * 16, 16)]
            dx = xv - plsc.load_gather(cxt, [sv])
            dy = yv - plsc.load_gather(cyt, [sv])
            dz = zv - plsc.load_gather(czt, [sv])
            s = dx * dx + dy * dy + dz * dz
            medv = plsc.load_gather(medt, [sv])
            fm = s <= medv
            ai = lane80 + sv * 4
            plsc.addupdate_scatter(acc, [ai], xv, mask=fm)
            plsc.addupdate_scatter(acc, [ai + 1], yv, mask=fm)
            plsc.addupdate_scatter(acc, [ai + 2], zv, mask=fm)
            plsc.addupdate_scatter(acc, [ai + 3], ones_f, mask=fm)

    stream_scan(xyzseg_pairs, s6_compute)
    merge_acc()

    pg = plo + lane
    lm = lane < PG
    i4 = jnp.where(lm, pg * 4, 0)
    fsx = plsc.load_gather(acc, [i4])
    fsy = plsc.load_gather(acc, [i4 + 1])
    fsz = plsc.load_gather(acc, [i4 + 2])
    fcn = plsc.load_gather(acc, [i4 + 3])
    cxv = plsc.load_gather(cxt, [pg])
    cyv = plsc.load_gather(cyt, [pg])
    czv = plsc.load_gather(czt, [pg])
    cnv = plsc.load_gather(cntt, [pg])
    den = jnp.maximum(fcn, 1.0)
    zf = jnp.zeros((16,), jnp.float32)
    has_f = fcn > 0.0
    has_c = cnv > 0.0
    bx = jnp.where(has_c, jnp.where(has_f, fsx / den, cxv), zf)
    by = jnp.where(has_c, jnp.where(has_f, fsy / den, cyv), zf)
    bz = jnp.where(has_c, jnp.where(has_f, fsz / den, czv), zf)
    for t in range(4):
        outrow[pl.ds(t * 16, 16)] = zf
    oidx = jnp.where(lm, lane * 3, 48)
    plsc.store_scatter(outrow, [oidx], bx, mask=lm)
    plsc.store_scatter(outrow, [oidx + 1], by, mask=lm)
    plsc.store_scatter(outrow, [oidx + 2], bz, mask=lm)
    pltpu.sync_copy(outrow, out_hbm.at[pl.ds(wid * 64, 64)])


@functools.lru_cache(maxsize=1)
def _build():
    mesh = plsc.VectorSubcoreMesh(core_axis_name="c", subcore_axis_name="s",
                                  num_cores=2, num_subcores=16)
    return pl.kernel(
        _sc_body,
        out_type=(
            jax.ShapeDtypeStruct((32 * 64,), jnp.float32),
            jax.ShapeDtypeStruct((B * N,), jnp.float32),
        ),
        mesh=mesh,
        compiler_params=pltpu.CompilerParams(needs_layout_passes=False),
        scratch_types=[
            pltpu.VMEM((C * 3,), jnp.float32),
            pltpu.VMEM((C * 3,), jnp.float32),
            pltpu.VMEM((C,), jnp.int32),
            pltpu.VMEM((C,), jnp.int32),
            pltpu.VMEM((C,), jnp.float32),
            pltpu.VMEM((C,), jnp.float32),
            pltpu.VMEM((HTOT,), jnp.float32),
            pltpu.VMEM((ACCT,), jnp.float32),
            pltpu.VMEM((256,), jnp.float32),
            pltpu.VMEM((PG * HPP,), jnp.float32),
            pltpu.VMEM((32,), jnp.int32),
            pltpu.VMEM((32,), jnp.float32),
            pltpu.VMEM((32,), jnp.float32),
            pltpu.VMEM((32,), jnp.float32),
            pltpu.VMEM((32,), jnp.float32),
            pltpu.VMEM((32,), jnp.float32),
            pltpu.VMEM((32,), jnp.int32),
            pltpu.VMEM((32,), jnp.float32),
            pltpu.VMEM((64,), jnp.float32),
            pltpu.VMEM_SHARED((16 * HPL,), jnp.float32),
            pltpu.VMEM_SHARED((16 * ACCW,), jnp.float32),
            pltpu.VMEM_SHARED((16 * 32,), jnp.int32),
            pltpu.VMEM_SHARED((16 * 32,), jnp.float32),
            pltpu.SemaphoreType.DMA,
            pltpu.SemaphoreType.DMA,
            pltpu.SemaphoreType.DMA,
            pltpu.SemaphoreType.DMA,
        ],
        name="pose_sc",
    )


def kernel(xyz, seg_labels):
    xyz_planar = jnp.transpose(xyz, (2, 0, 1)).reshape(-1)
    rows, _ = _build()(xyz_planar, seg_labels.reshape(-1))
    return rows.reshape(32, 64)[:, :27].reshape(B, 2, PG, 3).reshape(B, P, 3)
